# baseline (speedup 1.0000x reference)
import jax
import jax.numpy as jnp
from jax.experimental import pallas as pl
from jax.experimental.pallas import tpu as pltpu


def kernel(x, w_eff, b_eff, scale):
    bs, n_tok, d_model = x.shape
    t_dim, _, i_pad = w_eff.shape

    tk = 4
    n_k = n_tok // tk
    x4 = x.reshape(bs, n_k, tk, d_model)

    return pl.pallas_call(
        _matmul4_kernel,
        out_shape=jax.ShapeDtypeStruct((bs, i_pad), x.dtype),
        grid=(n_k,),
        in_specs=[
            pl.BlockSpec((bs, 1, tk, d_model), lambda k: (0, k, 0, 0)),
            pl.BlockSpec((tk, d_model, i_pad), lambda k: (k, 0, 0)),
            pl.BlockSpec((1, i_pad), lambda k: (0, 0)),
            pl.BlockSpec((1, i_pad), lambda k: (0, 0)),
        ],
        out_specs=pl.BlockSpec((bs, i_pad), lambda k: (0, 0)),
        compiler_params=pltpu.CompilerParams(
            dimension_semantics=("arbitrary",),
            vmem_limit_bytes=60 * 1024 * 1024,
        ),
    )(x4, w_eff, b_eff, scale)


def _matmul4_kernel(x_ref, w_ref, b_ref, s_ref, y_ref):
    k = pl.program_id(0)

    @pl.when(k == 0)
    def _():
        y_ref[...] = jnp.zeros_like(y_ref)

    bs = x_ref.shape[0]
    tk, d_model, i_pad = w_ref.shape
    y_ref[...] += jnp.dot(
        x_ref[...].astype(jnp.bfloat16).reshape(bs, tk * d_model),
        w_ref[...].astype(jnp.bfloat16).reshape(tk * d_model, i_pad),
        preferred_element_type=jnp.float32,
    )

    @pl.when(k == pl.num_programs(0) - 1)
    def _():
        y_ref[...] = (y_ref[...] + b_ref[...]) * s_ref[...]


# speedup vs baseline: 1.8081x; 1.8081x over previous
import jax
import jax.numpy as jnp
from jax.experimental import pallas as pl
from jax.experimental.pallas import tpu as pltpu


def _matmul_kernel(x_ref, w_ref, b_ref, s_ref, y_ref):
    k = pl.program_id(0)

    @pl.when(k == 0)
    def _():
        y_ref[...] = jnp.zeros_like(y_ref)

    bs = x_ref.shape[0]
    tk, d_model, i_pad = w_ref.shape
    y_ref[...] += jnp.dot(
        x_ref[...].astype(jnp.bfloat16).reshape(bs, tk * d_model),
        w_ref[...].astype(jnp.bfloat16).reshape(tk * d_model, i_pad),
        preferred_element_type=jnp.float32,
    )

    @pl.when(k == pl.num_programs(0) - 1)
    def _():
        y_ref[...] = (y_ref[...] + b_ref[...]) * s_ref[...]


def kernel(x, w_eff, b_eff, scale):
    bs, n_tok, d_model = x.shape
    t_dim, _, i_pad = w_eff.shape

    tk = 8
    while n_tok % tk:
        tk //= 2
    n_k = n_tok // tk

    return pl.pallas_call(
        _matmul_kernel,
        out_shape=jax.ShapeDtypeStruct((bs, i_pad), x.dtype),
        grid=(n_k,),
        in_specs=[
            pl.BlockSpec((bs, tk, d_model), lambda k: (0, k, 0)),
            pl.BlockSpec((tk, d_model, i_pad), lambda k: (k, 0, 0)),
            pl.BlockSpec((1, i_pad), lambda k: (0, 0)),
            pl.BlockSpec((1, i_pad), lambda k: (0, 0)),
        ],
        out_specs=pl.BlockSpec((bs, i_pad), lambda k: (0, 0)),
        compiler_params=pltpu.CompilerParams(
            dimension_semantics=("arbitrary",),
            vmem_limit_bytes=60 * 1024 * 1024,
        ),
    )(x, w_eff, b_eff, scale)
